# baseline (speedup 1.0000x reference)
import functools

import jax
import jax.numpy as jnp
from jax import lax
from jax.experimental import pallas as pl
from jax.experimental.pallas import tpu as pltpu
from jax.experimental.pallas import tpu_sc as plsc

N = 10000
E = 160000
D = 256
HD = 128
NP = 10240
EP = 163840
RB = 1024
NBLK = NP // RB
EROWS = EP // 128

NSC = 2
NTILE = 16

DEG_ROWS_PER_TILE = EP // (NSC * NTILE) // 128
SCAT_ROWS_PER_TILE = EP // NTILE // 128
NODE_ROWS_PER_TILE = NP // NTILE


def _deg_body(dst_hbm, ones_hbm, zeros_hbm, deg_out, deg_sh, dst_v, ones_v):
    c = lax.axis_index("c")
    s = lax.axis_index("s")
    w = c * NTILE + s
    nslc = pl.ds(s * NODE_ROWS_PER_TILE, NODE_ROWS_PER_TILE)
    pltpu.sync_copy(zeros_hbm.at[nslc], deg_sh.at[nslc])
    pltpu.sync_copy(
        dst_hbm.at[pl.ds(w * DEG_ROWS_PER_TILE, DEG_ROWS_PER_TILE)], dst_v)
    pltpu.sync_copy(ones_hbm, ones_v)
    plsc.subcore_barrier()

    def chunk(i, carry):
        pltpu.sync_copy(ones_v, deg_sh.at[dst_v.at[i]], add=True)
        return carry

    lax.fori_loop(0, DEG_ROWS_PER_TILE, chunk, 0)
    plsc.subcore_barrier()
    pltpu.sync_copy(deg_sh.at[nslc], deg_out.at[c, nslc])


@functools.cache
def _deg_kernel():
    return pl.kernel(
        _deg_body,
        out_type=jax.ShapeDtypeStruct((NSC, NP, 128), jnp.float32),
        mesh=plsc.VectorSubcoreMesh(core_axis_name="c", subcore_axis_name="s",
                                    num_cores=NSC, num_subcores=NTILE),
        scratch_types=[
            pltpu.VMEM_SHARED((NP, 128), jnp.float32),
            pltpu.VMEM((DEG_ROWS_PER_TILE, 128), jnp.int32),
            pltpu.VMEM((128, 128), jnp.float32),
        ],
    )


def _scatter_body(y_hbm, src_hbm, dst_hbm, zeros_hbm, s_out,
                  s_sh, src_v, dst_v, rows0_v, rows1_v,
                  gsem0, gsem1, gsem0b, gsem1b):
    c = lax.axis_index("c")
    s = lax.axis_index("s")
    nslc = pl.ds(s * NODE_ROWS_PER_TILE, NODE_ROWS_PER_TILE)
    pltpu.sync_copy(zeros_hbm.at[nslc], s_sh.at[nslc])
    plsc.subcore_barrier()

    half = SCAT_ROWS_PER_TILE // 2
    lo = pl.ds(0, 64)
    hi = pl.ds(64, 64)

    def gather(i, buf, sa, sb):
        pltpu.async_copy(y_hbm.at[src_v.at[i, lo]], buf.at[lo], sa)
        pltpu.async_copy(y_hbm.at[src_v.at[i, hi]], buf.at[hi], sb)

    def gwait(i, buf, sa, sb):
        pltpu.make_async_copy(y_hbm.at[src_v.at[i, lo]], buf.at[lo], sa).wait()
        pltpu.make_async_copy(y_hbm.at[src_v.at[i, hi]], buf.at[hi], sb).wait()

    def phase(p, carry):
        base = s * SCAT_ROWS_PER_TILE + p * half
        pltpu.sync_copy(src_hbm.at[c, pl.ds(base, half)], src_v)
        pltpu.sync_copy(dst_hbm.at[pl.ds(base, half)], dst_v)
        gather(0, rows0_v, gsem0, gsem0b)

        def pair(g, carry2):
            i0 = 2 * g
            i1 = 2 * g + 1
            gather(i1, rows1_v, gsem1, gsem1b)
            gwait(i0, rows0_v, gsem0, gsem0b)
            pltpu.sync_copy(rows0_v, s_sh.at[dst_v.at[i0]], add=True)

            @pl.when(g < half // 2 - 1)
            def _():
                gather(i0 + 2, rows0_v, gsem0, gsem0b)

            gwait(i1, rows1_v, gsem1, gsem1b)
            pltpu.sync_copy(rows1_v, s_sh.at[dst_v.at[i1]], add=True)
            return carry2

        lax.fori_loop(0, half // 2, pair, 0)
        return carry

    lax.fori_loop(0, 2, phase, 0)
    plsc.subcore_barrier()
    pltpu.sync_copy(s_sh.at[nslc],
                    s_out.at[pl.ds(c * NP + s * NODE_ROWS_PER_TILE,
                                   NODE_ROWS_PER_TILE)])


@functools.cache
def _scatter_kernel():
    return pl.kernel(
        _scatter_body,
        out_type=jax.ShapeDtypeStruct((NSC * NP, HD), jnp.float32),
        mesh=plsc.VectorSubcoreMesh(core_axis_name="c", subcore_axis_name="s",
                                    num_cores=NSC, num_subcores=NTILE),
        scratch_types=[
            pltpu.VMEM_SHARED((NP, HD), jnp.float32),
            pltpu.VMEM((SCAT_ROWS_PER_TILE // 2, 128), jnp.int32),
            pltpu.VMEM((SCAT_ROWS_PER_TILE // 2, 128), jnp.int32),
            pltpu.VMEM((128, HD), jnp.float32),
            pltpu.VMEM((128, HD), jnp.float32),
            pltpu.SemaphoreType.DMA,
            pltpu.SemaphoreType.DMA,
            pltpu.SemaphoreType.DMA,
            pltpu.SemaphoreType.DMA,
        ],
    )


def _matmul_body(x_ref, w_ref, xw_ref):
    xw_ref[...] = jnp.dot(x_ref[...], w_ref[...],
                          preferred_element_type=jnp.float32)


def _scale_body(xw_ref, degs_ref, y_ref):
    h = xw_ref[...]
    deg = 1.0 + jnp.sum(degs_ref[...], axis=(0, 2))
    dis = lax.rsqrt(deg)[:, None]
    y_ref[0] = h[:, :HD] * dis
    y_ref[1] = h[:, HD:] * dis


def _final_body(x_ref, sa_ref, sb_ref, ya_ref, yb_ref, degs_ref,
                bias_ref, w1_ref, w2_ref, w3_ref, out_ref):
    deg = 1.0 + jnp.sum(degs_ref[...], axis=(0, 2))
    dis = lax.rsqrt(deg)[:, None]
    bias = bias_ref[...]
    x = x_ref[...]
    ha = jax.nn.relu(dis * (sa_ref[...] + ya_ref[...]) + bias[0:1, :]) + x[:, :HD]
    hb = jax.nn.relu(dis * (sb_ref[...] + yb_ref[...]) + bias[1:2, :]) + x[:, HD:]
    w1 = w1_ref[...]
    h1 = jax.nn.relu(
        jnp.dot(ha, w1[:HD], preferred_element_type=jnp.float32)
        + jnp.dot(hb, w1[HD:], preferred_element_type=jnp.float32)
        + bias[2:3, :])
    h2 = jax.nn.relu(
        jnp.dot(h1, w2_ref[...], preferred_element_type=jnp.float32)
        + bias[3:4, :])
    h3 = (jnp.dot(h2, w3_ref[...], preferred_element_type=jnp.float32)
          + bias[4:5, :])
    out_ref[...] = h3[:, 0:1]


def kernel(x, edge_index, W_conv, b_conv, W1, b1, W2, b2, W3, b3):
    f32 = jnp.float32
    x_pad = jnp.pad(x, ((0, NP - N), (0, 0)))
    src = edge_index[0]
    dst = edge_index[1]
    src_p = jnp.concatenate([src, jnp.full((EP - E,), N, jnp.int32)])
    dst_p = jnp.concatenate([dst, jnp.full((EP - E,), N, jnp.int32)])
    src2 = jnp.stack([src_p, src_p + NP]).reshape(NSC, EROWS, 128)
    dst_r = dst_p.reshape(EROWS, 128)
    ones128 = jnp.zeros((128, 128), f32).at[:, 0].set(1.0)
    zeros128 = jnp.zeros((NP, HD), f32)

    degs = _deg_kernel()(dst_r, ones128, zeros128)

    wct = W_conv.T
    xw = pl.pallas_call(
        _matmul_body,
        grid=(NBLK,),
        in_specs=[
            pl.BlockSpec((RB, D), lambda i: (i, 0)),
            pl.BlockSpec((D, D), lambda i: (0, 0)),
        ],
        out_specs=pl.BlockSpec((RB, D), lambda i: (i, 0)),
        out_shape=jax.ShapeDtypeStruct((NP, D), f32),
    )(x_pad, wct)
    y3 = pl.pallas_call(
        _scale_body,
        grid=(NBLK,),
        in_specs=[
            pl.BlockSpec((RB, D), lambda i: (i, 0)),
            pl.BlockSpec((NSC, RB, 128), lambda i: (0, i, 0)),
        ],
        out_specs=pl.BlockSpec((NSC, RB, HD), lambda i: (0, i, 0)),
        out_shape=jax.ShapeDtypeStruct((NSC, NP, HD), f32),
    )(xw, degs)
    y_flat = y3.reshape(NSC * NP, HD)

    s_flat = _scatter_kernel()(y_flat, src2, dst_r, zeros128)

    bias = jnp.zeros((8, 128), f32)
    bias = bias.at[0, :].set(b_conv[:HD])
    bias = bias.at[1, :].set(b_conv[HD:])
    bias = bias.at[2, :32].set(b1)
    bias = bias.at[3, :32].set(b2)
    bias = bias.at[4, 0].set(b3[0])
    w1p = jnp.zeros((D, 128), f32).at[:, :32].set(W1.T)
    w2p = jnp.zeros((128, 128), f32).at[:32, :32].set(W2.T)
    w3p = jnp.zeros((128, 128), f32).at[:32, 0].set(W3[0])

    out = pl.pallas_call(
        _final_body,
        grid=(NBLK,),
        in_specs=[
            pl.BlockSpec((RB, D), lambda i: (i, 0)),
            pl.BlockSpec((RB, HD), lambda i: (i, 0)),
            pl.BlockSpec((RB, HD), lambda i: (i + NBLK, 0)),
            pl.BlockSpec((RB, HD), lambda i: (i, 0)),
            pl.BlockSpec((RB, HD), lambda i: (i + NBLK, 0)),
            pl.BlockSpec((NSC, RB, 128), lambda i: (0, i, 0)),
            pl.BlockSpec((8, 128), lambda i: (0, 0)),
            pl.BlockSpec((D, 128), lambda i: (0, 0)),
            pl.BlockSpec((128, 128), lambda i: (0, 0)),
            pl.BlockSpec((128, 128), lambda i: (0, 0)),
        ],
        out_specs=pl.BlockSpec((RB, 1), lambda i: (i, 0)),
        out_shape=jax.ShapeDtypeStruct((NP, 1), f32),
    )(x_pad, s_flat, s_flat, y_flat, y_flat, degs, bias, w1p, w2p, w3p)
    return out[:N]


# speedup vs baseline: 1.2072x; 1.2072x over previous
import functools

import jax
import jax.numpy as jnp
from jax import lax
from jax.experimental import pallas as pl
from jax.experimental.pallas import tpu as pltpu
from jax.experimental.pallas import tpu_sc as plsc

N = 10000
E = 160000
D = 256
HD = 128
NP = 10240
EP = 163840
RB = 1024
NBLK = NP // RB
EROWS = EP // 128

NSC = 2
NTILE = 16

DEG_ROWS_PER_TILE = EP // (NSC * NTILE) // 128
SCAT_ROWS_PER_TILE = EP // NTILE // 128
NODE_ROWS_PER_TILE = NP // NTILE


def _deg_body(dst_hbm, ones_hbm, zeros_hbm, deg_out, deg_sh, dst_v, ones_v):
    c = lax.axis_index("c")
    s = lax.axis_index("s")
    w = c * NTILE + s
    nslc = pl.ds(s * NODE_ROWS_PER_TILE, NODE_ROWS_PER_TILE)
    pltpu.sync_copy(zeros_hbm.at[nslc], deg_sh.at[nslc])
    pltpu.sync_copy(
        dst_hbm.at[pl.ds(w * DEG_ROWS_PER_TILE, DEG_ROWS_PER_TILE)], dst_v)
    pltpu.sync_copy(ones_hbm, ones_v)
    plsc.subcore_barrier()

    def chunk(i, carry):
        pltpu.sync_copy(ones_v, deg_sh.at[dst_v.at[i]], add=True)
        return carry

    lax.fori_loop(0, DEG_ROWS_PER_TILE, chunk, 0)
    plsc.subcore_barrier()
    pltpu.sync_copy(deg_sh.at[nslc], deg_out.at[c, nslc])


@functools.cache
def _deg_kernel():
    return pl.kernel(
        _deg_body,
        out_type=jax.ShapeDtypeStruct((NSC, NP, 128), jnp.float32),
        mesh=plsc.VectorSubcoreMesh(core_axis_name="c", subcore_axis_name="s",
                                    num_cores=NSC, num_subcores=NTILE),
        scratch_types=[
            pltpu.VMEM_SHARED((NP, 128), jnp.float32),
            pltpu.VMEM((DEG_ROWS_PER_TILE, 128), jnp.int32),
            pltpu.VMEM((128, 128), jnp.float32),
        ],
    )


def _scatter_body(y_hbm, src_hbm, dst_hbm, zeros_hbm, s_out,
                  s_sh, src_v, dst_v, rows0_v, rows1_v,
                  gsem0, gsem1, gsem0b, gsem1b):
    c = lax.axis_index("c")
    s = lax.axis_index("s")
    nslc = pl.ds(s * NODE_ROWS_PER_TILE, NODE_ROWS_PER_TILE)
    pltpu.sync_copy(zeros_hbm.at[nslc], s_sh.at[nslc])
    plsc.subcore_barrier()

    half = SCAT_ROWS_PER_TILE // 2
    lo = pl.ds(0, 64)
    hi = pl.ds(64, 64)

    def gather(i, buf, sa, sb):
        pltpu.async_copy(y_hbm.at[src_v.at[i, lo]], buf.at[lo], sa)
        pltpu.async_copy(y_hbm.at[src_v.at[i, hi]], buf.at[hi], sb)

    def gwait(i, buf, sa, sb):
        pltpu.make_async_copy(y_hbm.at[src_v.at[i, lo]], buf.at[lo], sa).wait()
        pltpu.make_async_copy(y_hbm.at[src_v.at[i, hi]], buf.at[hi], sb).wait()

    def phase(p, carry):
        base = s * SCAT_ROWS_PER_TILE + p * half
        pltpu.sync_copy(src_hbm.at[c, pl.ds(base, half)], src_v)
        pltpu.sync_copy(dst_hbm.at[pl.ds(base, half)], dst_v)
        gather(0, rows0_v, gsem0, gsem0b)

        def pair(g, carry2):
            i0 = 2 * g
            i1 = 2 * g + 1
            gather(i1, rows1_v, gsem1, gsem1b)
            gwait(i0, rows0_v, gsem0, gsem0b)
            pltpu.sync_copy(rows0_v, s_sh.at[dst_v.at[i0]], add=True)

            @pl.when(g < half // 2 - 1)
            def _():
                gather(i0 + 2, rows0_v, gsem0, gsem0b)

            gwait(i1, rows1_v, gsem1, gsem1b)
            pltpu.sync_copy(rows1_v, s_sh.at[dst_v.at[i1]], add=True)
            return carry2

        lax.fori_loop(0, half // 2, pair, 0)
        return carry

    lax.fori_loop(0, 2, phase, 0)
    plsc.subcore_barrier()
    pltpu.sync_copy(s_sh.at[nslc],
                    s_out.at[pl.ds(c * NP + s * NODE_ROWS_PER_TILE,
                                   NODE_ROWS_PER_TILE)])


@functools.cache
def _scatter_kernel():
    return pl.kernel(
        _scatter_body,
        out_type=jax.ShapeDtypeStruct((NSC * NP, HD), jnp.float32),
        mesh=plsc.VectorSubcoreMesh(core_axis_name="c", subcore_axis_name="s",
                                    num_cores=NSC, num_subcores=NTILE),
        scratch_types=[
            pltpu.VMEM_SHARED((NP, HD), jnp.float32),
            pltpu.VMEM((SCAT_ROWS_PER_TILE // 2, 128), jnp.int32),
            pltpu.VMEM((SCAT_ROWS_PER_TILE // 2, 128), jnp.int32),
            pltpu.VMEM((128, HD), jnp.float32),
            pltpu.VMEM((128, HD), jnp.float32),
            pltpu.SemaphoreType.DMA,
            pltpu.SemaphoreType.DMA,
            pltpu.SemaphoreType.DMA,
            pltpu.SemaphoreType.DMA,
        ],
    )


def _matmul_scale_body(x_ref, w_ref, degs_ref, y_ref, dis_ref):
    i = pl.program_id(0)
    h = jnp.dot(x_ref[...], w_ref[...], preferred_element_type=jnp.float32)
    deg = 1.0 + jnp.sum(degs_ref[...], axis=(0, 2))
    dis = lax.rsqrt(deg)[:, None]
    rowid = lax.broadcasted_iota(jnp.int32, (RB, 1), 0) + i * RB
    dis_m = jnp.where(rowid < N, dis, 0.0)
    y_ref[0] = h[:, :HD] * dis_m
    y_ref[1] = h[:, HD:] * dis_m
    dis_ref[...] = dis


def _final_body(x_ref, sa_ref, sb_ref, ya_ref, yb_ref, dis_ref,
                bias_ref, w1_ref, w2_ref, w3_ref, out_ref):
    dis = dis_ref[...]
    bias = bias_ref[...]
    x = x_ref[...]
    ha = jax.nn.relu(dis * (sa_ref[...] + ya_ref[...]) + bias[0:1, :]) + x[:, :HD]
    hb = jax.nn.relu(dis * (sb_ref[...] + yb_ref[...]) + bias[1:2, :]) + x[:, HD:]
    w1 = w1_ref[...]
    h1 = jax.nn.relu(
        jnp.dot(ha, w1[:HD], preferred_element_type=jnp.float32)
        + jnp.dot(hb, w1[HD:], preferred_element_type=jnp.float32)
        + bias[2:3, :])
    h2 = jax.nn.relu(
        jnp.dot(h1, w2_ref[...], preferred_element_type=jnp.float32)
        + bias[3:4, :])
    h3 = (jnp.dot(h2, w3_ref[...], preferred_element_type=jnp.float32)
          + bias[4:5, :])
    out_ref[...] = h3[:, 0:1]


def kernel(x, edge_index, W_conv, b_conv, W1, b1, W2, b2, W3, b3):
    f32 = jnp.float32
    src = edge_index[0]
    dst = edge_index[1]
    src_p = jnp.concatenate([src, jnp.full((EP - E,), N, jnp.int32)])
    dst_p = jnp.concatenate([dst, jnp.full((EP - E,), N, jnp.int32)])
    src2 = jnp.stack([src_p, src_p + NP]).reshape(NSC, EROWS, 128)
    dst_r = dst_p.reshape(EROWS, 128)
    ones128 = jnp.zeros((128, 128), f32).at[:, 0].set(1.0)
    zeros128 = jnp.zeros((NP, HD), f32)

    degs = _deg_kernel()(dst_r, ones128, zeros128)

    wct = W_conv.T
    y3 = pl.pallas_call(
        _matmul_scale_body,
        grid=(NBLK,),
        in_specs=[
            pl.BlockSpec((RB, D), lambda i: (i, 0)),
            pl.BlockSpec((D, D), lambda i: (0, 0)),
            pl.BlockSpec((NSC, RB, 128), lambda i: (0, i, 0)),
        ],
        out_specs=[
            pl.BlockSpec((NSC, RB, HD), lambda i: (0, i, 0)),
            pl.BlockSpec((RB, 1), lambda i: (i, 0)),
        ],
        out_shape=[
            jax.ShapeDtypeStruct((NSC, NP, HD), f32),
            jax.ShapeDtypeStruct((NP, 1), f32),
        ],
    )(x, wct, degs)
    y3, dis_col = y3
    y_flat = y3.reshape(NSC * NP, HD)

    s_flat = _scatter_kernel()(y_flat, src2, dst_r, zeros128)

    bias = jnp.zeros((8, 128), f32)
    bias = bias.at[0, :].set(b_conv[:HD])
    bias = bias.at[1, :].set(b_conv[HD:])
    bias = bias.at[2, :32].set(b1)
    bias = bias.at[3, :32].set(b2)
    bias = bias.at[4, 0].set(b3[0])
    w1p = jnp.zeros((D, 128), f32).at[:, :32].set(W1.T)
    w2p = jnp.zeros((128, 128), f32).at[:32, :32].set(W2.T)
    w3p = jnp.zeros((128, 128), f32).at[:32, 0].set(W3[0])

    out = pl.pallas_call(
        _final_body,
        grid=(NBLK,),
        in_specs=[
            pl.BlockSpec((RB, D), lambda i: (i, 0)),
            pl.BlockSpec((RB, HD), lambda i: (i, 0)),
            pl.BlockSpec((RB, HD), lambda i: (i + NBLK, 0)),
            pl.BlockSpec((RB, HD), lambda i: (i, 0)),
            pl.BlockSpec((RB, HD), lambda i: (i + NBLK, 0)),
            pl.BlockSpec((RB, 1), lambda i: (i, 0)),
            pl.BlockSpec((8, 128), lambda i: (0, 0)),
            pl.BlockSpec((D, 128), lambda i: (0, 0)),
            pl.BlockSpec((128, 128), lambda i: (0, 0)),
            pl.BlockSpec((128, 128), lambda i: (0, 0)),
        ],
        out_specs=pl.BlockSpec((RB, 1), lambda i: (i, 0)),
        out_shape=jax.ShapeDtypeStruct((NP, 1), f32),
    )(x, s_flat, s_flat, y_flat, y_flat, dis_col, bias, w1p, w2p, w3p)
    return out[:N]


# speedup vs baseline: 1.2189x; 1.0096x over previous
import functools

import jax
import jax.numpy as jnp
from jax import lax
from jax.experimental import pallas as pl
from jax.experimental.pallas import tpu as pltpu
from jax.experimental.pallas import tpu_sc as plsc

N = 10000
E = 160000
D = 256
HD = 128
NP = 10240
EP = 163840
RB = 2048
NBLK = NP // RB
EROWS = EP // 128

NSC = 2
NTILE = 16

DEG_ROWS_PER_TILE = EP // (NSC * NTILE) // 128
SCAT_ROWS_PER_TILE = EP // NTILE // 128
NODE_ROWS_PER_TILE = NP // NTILE


def _deg_body(dst_hbm, ones_hbm, zeros_hbm, deg_out, deg_sh, dst_v, ones_v):
    c = lax.axis_index("c")
    s = lax.axis_index("s")
    w = c * NTILE + s
    nslc = pl.ds(s * NODE_ROWS_PER_TILE, NODE_ROWS_PER_TILE)
    pltpu.sync_copy(zeros_hbm.at[nslc], deg_sh.at[nslc])
    pltpu.sync_copy(
        dst_hbm.at[pl.ds(w * DEG_ROWS_PER_TILE, DEG_ROWS_PER_TILE)], dst_v)
    pltpu.sync_copy(ones_hbm, ones_v)
    plsc.subcore_barrier()

    def chunk(i, carry):
        pltpu.sync_copy(ones_v, deg_sh.at[dst_v.at[i]], add=True)
        return carry

    lax.fori_loop(0, DEG_ROWS_PER_TILE, chunk, 0)
    plsc.subcore_barrier()
    pltpu.sync_copy(deg_sh.at[nslc], deg_out.at[c, nslc])


@functools.cache
def _deg_kernel():
    return pl.kernel(
        _deg_body,
        out_type=jax.ShapeDtypeStruct((NSC, NP, 128), jnp.float32),
        mesh=plsc.VectorSubcoreMesh(core_axis_name="c", subcore_axis_name="s",
                                    num_cores=NSC, num_subcores=NTILE),
        scratch_types=[
            pltpu.VMEM_SHARED((NP, 128), jnp.float32),
            pltpu.VMEM((DEG_ROWS_PER_TILE, 128), jnp.int32),
            pltpu.VMEM((128, 128), jnp.float32),
        ],
    )


def _scatter_body(y_hbm, src_hbm, dst_hbm, zeros_hbm, s_out,
                  s_sh, src_v, dst_v, rows0_v, rows1_v,
                  gsem0, gsem1, gsem0b, gsem1b):
    c = lax.axis_index("c")
    s = lax.axis_index("s")
    nslc = pl.ds(s * NODE_ROWS_PER_TILE, NODE_ROWS_PER_TILE)
    pltpu.sync_copy(zeros_hbm.at[nslc], s_sh.at[nslc])
    plsc.subcore_barrier()

    half = SCAT_ROWS_PER_TILE // 2
    lo = pl.ds(0, 64)
    hi = pl.ds(64, 64)

    def gather(i, buf, sa, sb):
        pltpu.async_copy(y_hbm.at[src_v.at[i, lo]], buf.at[lo], sa)
        pltpu.async_copy(y_hbm.at[src_v.at[i, hi]], buf.at[hi], sb)

    def gwait(i, buf, sa, sb):
        pltpu.make_async_copy(y_hbm.at[src_v.at[i, lo]], buf.at[lo], sa).wait()
        pltpu.make_async_copy(y_hbm.at[src_v.at[i, hi]], buf.at[hi], sb).wait()

    def phase(p, carry):
        base = s * SCAT_ROWS_PER_TILE + p * half
        pltpu.sync_copy(src_hbm.at[c, pl.ds(base, half)], src_v)
        pltpu.sync_copy(dst_hbm.at[pl.ds(base, half)], dst_v)
        gather(0, rows0_v, gsem0, gsem0b)

        def pair(g, carry2):
            i0 = 2 * g
            i1 = 2 * g + 1
            gather(i1, rows1_v, gsem1, gsem1b)
            gwait(i0, rows0_v, gsem0, gsem0b)
            pltpu.sync_copy(rows0_v, s_sh.at[dst_v.at[i0]], add=True)

            @pl.when(g < half // 2 - 1)
            def _():
                gather(i0 + 2, rows0_v, gsem0, gsem0b)

            gwait(i1, rows1_v, gsem1, gsem1b)
            pltpu.sync_copy(rows1_v, s_sh.at[dst_v.at[i1]], add=True)
            return carry2

        lax.fori_loop(0, half // 2, pair, 0)
        return carry

    lax.fori_loop(0, 2, phase, 0)
    plsc.subcore_barrier()
    pltpu.sync_copy(s_sh.at[nslc],
                    s_out.at[pl.ds(c * NP + s * NODE_ROWS_PER_TILE,
                                   NODE_ROWS_PER_TILE)])


@functools.cache
def _scatter_kernel():
    return pl.kernel(
        _scatter_body,
        out_type=jax.ShapeDtypeStruct((NSC * NP, HD), jnp.float32),
        mesh=plsc.VectorSubcoreMesh(core_axis_name="c", subcore_axis_name="s",
                                    num_cores=NSC, num_subcores=NTILE),
        scratch_types=[
            pltpu.VMEM_SHARED((NP, HD), jnp.float32),
            pltpu.VMEM((SCAT_ROWS_PER_TILE // 2, 128), jnp.int32),
            pltpu.VMEM((SCAT_ROWS_PER_TILE // 2, 128), jnp.int32),
            pltpu.VMEM((128, HD), jnp.float32),
            pltpu.VMEM((128, HD), jnp.float32),
            pltpu.SemaphoreType.DMA,
            pltpu.SemaphoreType.DMA,
            pltpu.SemaphoreType.DMA,
            pltpu.SemaphoreType.DMA,
        ],
    )


def _matmul_scale_body(x_ref, w_ref, degs_ref, y_ref, dis_ref):
    i = pl.program_id(0)
    h = jnp.dot(x_ref[...], w_ref[...], preferred_element_type=jnp.float32)
    deg = 1.0 + jnp.sum(degs_ref[...], axis=(0, 2))
    dis = lax.rsqrt(deg)[:, None]
    rowid = lax.broadcasted_iota(jnp.int32, (RB, 1), 0) + i * RB
    dis_m = jnp.where(rowid < N, dis, 0.0)
    y_ref[0] = h[:, :HD] * dis_m
    y_ref[1] = h[:, HD:] * dis_m
    dis_ref[...] = dis


def _final_body(x_ref, sa_ref, sb_ref, ya_ref, yb_ref, dis_ref,
                bias_ref, w1_ref, w2_ref, w3_ref, out_ref):
    dis = dis_ref[...]
    bias = bias_ref[...]
    x = x_ref[...]
    ha = jax.nn.relu(dis * (sa_ref[...] + ya_ref[...]) + bias[0:1, :]) + x[:, :HD]
    hb = jax.nn.relu(dis * (sb_ref[...] + yb_ref[...]) + bias[1:2, :]) + x[:, HD:]
    w1 = w1_ref[...]
    h1 = jax.nn.relu(
        jnp.dot(ha, w1[:HD], preferred_element_type=jnp.float32)
        + jnp.dot(hb, w1[HD:], preferred_element_type=jnp.float32)
        + bias[2:3, :])
    h2 = jax.nn.relu(
        jnp.dot(h1, w2_ref[...], preferred_element_type=jnp.float32)
        + bias[3:4, :])
    h3 = (jnp.dot(h2, w3_ref[...], preferred_element_type=jnp.float32)
          + bias[4:5, :])
    out_ref[...] = h3[:, 0:1]


def kernel(x, edge_index, W_conv, b_conv, W1, b1, W2, b2, W3, b3):
    f32 = jnp.float32
    src = edge_index[0]
    dst = edge_index[1]
    src_p = jnp.concatenate([src, jnp.full((EP - E,), N, jnp.int32)])
    dst_p = jnp.concatenate([dst, jnp.full((EP - E,), N, jnp.int32)])
    src2 = jnp.stack([src_p, src_p + NP]).reshape(NSC, EROWS, 128)
    dst_r = dst_p.reshape(EROWS, 128)
    ones128 = jnp.zeros((128, 128), f32).at[:, 0].set(1.0)
    zeros128 = jnp.zeros((NP, HD), f32)

    degs = _deg_kernel()(dst_r, ones128, zeros128)

    wct = W_conv.T
    y3 = pl.pallas_call(
        _matmul_scale_body,
        grid=(NBLK,),
        in_specs=[
            pl.BlockSpec((RB, D), lambda i: (i, 0)),
            pl.BlockSpec((D, D), lambda i: (0, 0)),
            pl.BlockSpec((NSC, RB, 128), lambda i: (0, i, 0)),
        ],
        out_specs=[
            pl.BlockSpec((NSC, RB, HD), lambda i: (0, i, 0)),
            pl.BlockSpec((RB, 1), lambda i: (i, 0)),
        ],
        out_shape=[
            jax.ShapeDtypeStruct((NSC, NP, HD), f32),
            jax.ShapeDtypeStruct((NP, 1), f32),
        ],
    )(x, wct, degs)
    y3, dis_col = y3
    y_flat = y3.reshape(NSC * NP, HD)

    s_flat = _scatter_kernel()(y_flat, src2, dst_r, zeros128)

    bias = jnp.zeros((8, 128), f32)
    bias = bias.at[0, :].set(b_conv[:HD])
    bias = bias.at[1, :].set(b_conv[HD:])
    bias = bias.at[2, :32].set(b1)
    bias = bias.at[3, :32].set(b2)
    bias = bias.at[4, 0].set(b3[0])
    w1p = jnp.zeros((D, 128), f32).at[:, :32].set(W1.T)
    w2p = jnp.zeros((128, 128), f32).at[:32, :32].set(W2.T)
    w3p = jnp.zeros((128, 128), f32).at[:32, 0].set(W3[0])

    out = pl.pallas_call(
        _final_body,
        grid=(NBLK,),
        in_specs=[
            pl.BlockSpec((RB, D), lambda i: (i, 0)),
            pl.BlockSpec((RB, HD), lambda i: (i, 0)),
            pl.BlockSpec((RB, HD), lambda i: (i + NBLK, 0)),
            pl.BlockSpec((RB, HD), lambda i: (i, 0)),
            pl.BlockSpec((RB, HD), lambda i: (i + NBLK, 0)),
            pl.BlockSpec((RB, 1), lambda i: (i, 0)),
            pl.BlockSpec((8, 128), lambda i: (0, 0)),
            pl.BlockSpec((D, 128), lambda i: (0, 0)),
            pl.BlockSpec((128, 128), lambda i: (0, 0)),
            pl.BlockSpec((128, 128), lambda i: (0, 0)),
        ],
        out_specs=pl.BlockSpec((RB, 1), lambda i: (i, 0)),
        out_shape=jax.ShapeDtypeStruct((NP, 1), f32),
    )(x, s_flat, s_flat, y_flat, y_flat, dis_col, bias, w1p, w2p, w3p)
    return out[:N]
